# baseline (speedup 1.0000x reference)
import functools

import jax
import jax.numpy as jnp
from jax import lax
from jax.experimental import pallas as pl
from jax.experimental.pallas import tpu as pltpu
from jax.experimental.pallas import tpu_sc as plsc

_M = 100000
_B = 1024
_DK = 64
_DOUT = 64
_THRESH = 0.1
_BM = 512
_NB = _M // _BM + 1
_LAST_OFF = _M - _BM

_NEG_HUGE = -3.0e38


def _score_body(XT, YT, xkb, ykb, nt, colv):
    dn = (((0,), (0,)), ((), ()))
    kq_x = lax.dot_general(xkb, XT, dn, preferred_element_type=jnp.float32)
    kq_y = lax.dot_general(ykb, YT, dn, preferred_element_type=jnp.float32)
    ones = jnp.ones((_DK, 1), jnp.float32)
    yn_col = lax.dot_general(ykb * ykb, ones, dn,
                             preferred_element_type=jnp.float32,
                             precision=lax.Precision.HIGHEST)
    ones1 = jnp.ones((1, _DK), jnp.float32)
    y2_row = lax.dot_general(ones1, YT * YT, (((1,), (0,)), ((), ())),
                             preferred_element_type=jnp.float32,
                             precision=lax.Precision.HIGHEST)
    ynh_col = 0.5 * yn_col
    y2h_row = 0.5 * y2_row - (0.5 * _THRESH)
    gate = (kq_y - ynh_col > y2h_row) & (kq_x > 0.0)
    scores = jnp.where(gate, (kq_x + kq_y) + nt, nt)
    blk_max = jnp.max(scores, axis=0, keepdims=True)
    cand = jnp.where(scores == blk_max, colv, 3.0e38)
    blk_arg = jnp.min(cand, axis=0, keepdims=True).astype(jnp.int32)
    blk_any = jnp.any(gate)
    return blk_max, blk_arg, blk_any


def _score_kernel(XT_ref, YT_ref, xkov_ref, ykov_ref, xkT_hbm, ykT_hbm, nT_hbm,
                  idx_out, flag_out,
                  runmax, runidx, flagacc,
                  xb0, xb1, yb0, yb1, nb0, nb1, sem0, sem1):
    i = pl.program_id(0)

    def _issue(k, xb, yb, nb, sem):
        k = jnp.asarray(k, jnp.int32)
        off = pl.multiple_of(jnp.minimum(k * _BM, _LAST_OFF), 8)

        @pl.when(k < _NB - 1)
        def _keys():
            pltpu.make_async_copy(xkT_hbm.at[:, pl.ds(k * _BM, _BM)], xb,
                                  sem).start()
            pltpu.make_async_copy(ykT_hbm.at[:, pl.ds(k * _BM, _BM)], yb,
                                  sem).start()

        pltpu.make_async_copy(nT_hbm.at[pl.ds(off, _BM), :], nb, sem).start()

    def _wait(k, xb, yb, nb, sem):
        @pl.when(k < _NB - 1)
        def _keys():
            pltpu.make_async_copy(xkT_hbm.at[:, pl.ds(0, _BM)], xb, sem).wait()
            pltpu.make_async_copy(ykT_hbm.at[:, pl.ds(0, _BM)], yb, sem).wait()

        pltpu.make_async_copy(nT_hbm.at[pl.ds(0, _BM), :], nb, sem).wait()

    XT = XT_ref[...]
    YT = YT_ref[...]

    def _merge(blk_max, blk_arg, blk_any):
        better = blk_max > runmax[...]
        runmax[...] = jnp.where(better, blk_max, runmax[...])
        runidx[...] = jnp.where(better, blk_arg, runidx[...])
        flagacc[...] = jnp.maximum(flagacc[...], blk_any.astype(jnp.float32))

    @pl.when(i == 0)
    def _init():
        runmax[...] = jnp.full((1, _B), _NEG_HUGE, jnp.float32)
        runidx[...] = jnp.zeros((1, _B), jnp.int32)
        flagacc[...] = jnp.zeros((1, 1), jnp.float32)
        _issue(0, xb0, yb0, nb0, sem0)

    @pl.when(i == _NB - 1)
    def _ov_keys():
        xb0[...] = xkov_ref[...]
        yb0[...] = ykov_ref[...]

    colv = lax.broadcasted_iota(jnp.int32, (_BM, 1), 0).astype(jnp.float32)
    off = jnp.minimum(i * _BM, _LAST_OFF)

    def _step(xb, yb, nb, sem, xbn, ybn, nbn, semn):
        @pl.when(i + 1 < _NB)
        def _pf():
            _issue(i + 1, xbn, ybn, nbn, semn)

        _wait(i, xb, yb, nb, sem)
        blk_max, blk_arg, blk_any = _score_body(XT, YT, xb[...], yb[...],
                                                nb[...], colv)
        _merge(blk_max, blk_arg + off, blk_any)

    @pl.when(i % 2 == 0)
    def _even():
        _step(xb0, yb0, nb0, sem0, xb1, yb1, nb1, sem1)

    @pl.when(i % 2 == 1)
    def _odd():
        _step(xb1, yb1, nb1, sem1, xb0, yb0, nb0, sem0)

    @pl.when(i == _NB - 1)
    def _fin():
        idx_out[...] = runidx[...]
        flag_out[...] = flagacc[...]


def _scores_argmax(X, Y, x_keys, y_keys, noise):
    XT = X.T
    YT = Y.T
    xkT = x_keys.T
    ykT = y_keys.T
    nT = noise.T
    xk_ov = lax.slice(x_keys, (_LAST_OFF, 0), (_M, _DK)).T
    yk_ov = lax.slice(y_keys, (_LAST_OFF, 0), (_M, _DK)).T
    idx, flag = pl.pallas_call(
        _score_kernel,
        grid=(_NB,),
        in_specs=[
            pl.BlockSpec((_DK, _B), lambda i: (0, 0)),
            pl.BlockSpec((_DK, _B), lambda i: (0, 0)),
            pl.BlockSpec((_DK, _BM), lambda i: (0, 0)),
            pl.BlockSpec((_DK, _BM), lambda i: (0, 0)),
            pl.BlockSpec(memory_space=pl.ANY),
            pl.BlockSpec(memory_space=pl.ANY),
            pl.BlockSpec(memory_space=pl.ANY),
        ],
        out_specs=[
            pl.BlockSpec((1, _B), lambda i: (0, 0)),
            pl.BlockSpec((1, 1), lambda i: (0, 0)),
        ],
        out_shape=[
            jax.ShapeDtypeStruct((1, _B), jnp.int32),
            jax.ShapeDtypeStruct((1, 1), jnp.float32),
        ],
        scratch_shapes=[
            pltpu.VMEM((1, _B), jnp.float32),
            pltpu.VMEM((1, _B), jnp.int32),
            pltpu.VMEM((1, 1), jnp.float32),
            pltpu.VMEM((_DK, _BM), jnp.float32),
            pltpu.VMEM((_DK, _BM), jnp.float32),
            pltpu.VMEM((_DK, _BM), jnp.float32),
            pltpu.VMEM((_DK, _BM), jnp.float32),
            pltpu.VMEM((_BM, _B), jnp.float32),
            pltpu.VMEM((_BM, _B), jnp.float32),
            pltpu.SemaphoreType.DMA,
            pltpu.SemaphoreType.DMA,
        ],
    )(XT, YT, xk_ov, yk_ov, xkT, ykT, nT)
    return idx, flag


def _make_sc_gather():
    info = plsc.get_sparse_core_info()
    nw = info.num_cores * info.num_subcores
    b_per_w = _B // nw
    mesh = plsc.VectorSubcoreMesh(core_axis_name="c", subcore_axis_name="s")

    @functools.partial(
        pl.kernel, mesh=mesh,
        out_type=jax.ShapeDtypeStruct((_B, 2 * _DOUT), jnp.float32),
        scratch_types=[
            pltpu.VMEM((b_per_w,), jnp.int32),
            pltpu.VMEM((b_per_w, 2 * _DOUT), jnp.float32),
            pltpu.SemaphoreType.DMA,
        ],
    )
    def _gather(table_hbm, idx_hbm, out_hbm, idx_v, rows_v, sem):
        wid = lax.axis_index("s") * info.num_cores + lax.axis_index("c")
        base = wid * b_per_w
        pltpu.sync_copy(idx_hbm.at[pl.ds(base, b_per_w)], idx_v)
        pltpu.async_copy(table_hbm.at[idx_v], rows_v, sem).wait()
        pltpu.sync_copy(rows_v, out_hbm.at[pl.ds(base, b_per_w)])

    return _gather


def kernel(X, Y, x_keys, y_keys, values, noise):
    idx, flag = _scores_argmax(X, Y, x_keys, y_keys, noise)
    idx = idx.reshape(_B)
    gather = _make_sc_gather()
    pair = gather(values.reshape(_M // 2, 2 * _DOUT), jnp.right_shift(idx, 1))
    x_hat = jnp.where((idx & 1)[:, None] == 1, pair[:, _DOUT:], pair[:, :_DOUT])
    return jnp.where(flag[0, 0] > 0.0, x_hat, jnp.zeros_like(x_hat))


# speedup vs baseline: 1.0943x; 1.0943x over previous
import functools

import jax
import jax.numpy as jnp
from jax import lax
from jax.experimental import pallas as pl
from jax.experimental.pallas import tpu as pltpu
from jax.experimental.pallas import tpu_sc as plsc

_M = 100000
_B = 1024
_DK = 64
_DOUT = 64
_THRESH = 0.1
_BM = 1536
_NB = _M // _BM + 1
_LAST_OFF = _M - _BM

_NEG_HUGE = -3.0e38


def _score_body(XT, YT, xkb, ykb, nt, colv):
    dn = (((0,), (0,)), ((), ()))
    kq_x = lax.dot_general(xkb, XT, dn, preferred_element_type=jnp.float32)
    kq_y = lax.dot_general(ykb, YT, dn, preferred_element_type=jnp.float32)
    ones = jnp.ones((_DK, 1), jnp.float32)
    yn_col = lax.dot_general(ykb * ykb, ones, dn,
                             preferred_element_type=jnp.float32,
                             precision=lax.Precision.HIGHEST)
    ones1 = jnp.ones((1, _DK), jnp.float32)
    y2_row = lax.dot_general(ones1, YT * YT, (((1,), (0,)), ((), ())),
                             preferred_element_type=jnp.float32,
                             precision=lax.Precision.HIGHEST)
    ynh_col = 0.5 * yn_col
    y2h_row = 0.5 * y2_row - (0.5 * _THRESH)
    gate = (kq_y - ynh_col > y2h_row) & (kq_x > 0.0)
    scores = jnp.where(gate, (kq_x + kq_y) + nt, nt)
    blk_max = jnp.max(scores, axis=0, keepdims=True)
    cand = jnp.where(scores == blk_max, colv, 3.0e38)
    blk_arg = jnp.min(cand, axis=0, keepdims=True).astype(jnp.int32)
    blk_any = jnp.any(gate)
    return blk_max, blk_arg, blk_any


def _score_kernel(XT_ref, YT_ref, xkov_ref, ykov_ref, xkT_hbm, ykT_hbm, nT_hbm,
                  idx_out, flag_out,
                  runmax, runidx, flagacc,
                  xb0, xb1, yb0, yb1, nb0, nb1, sem0, sem1):
    i = pl.program_id(0)

    def _issue(k, xb, yb, nb, sem):
        k = jnp.asarray(k, jnp.int32)
        off = pl.multiple_of(jnp.minimum(k * _BM, _LAST_OFF), 8)

        @pl.when(k < _NB - 1)
        def _keys():
            pltpu.make_async_copy(xkT_hbm.at[:, pl.ds(k * _BM, _BM)], xb,
                                  sem).start()
            pltpu.make_async_copy(ykT_hbm.at[:, pl.ds(k * _BM, _BM)], yb,
                                  sem).start()

        pltpu.make_async_copy(nT_hbm.at[pl.ds(off, _BM), :], nb, sem).start()

    def _wait(k, xb, yb, nb, sem):
        @pl.when(k < _NB - 1)
        def _keys():
            pltpu.make_async_copy(xkT_hbm.at[:, pl.ds(0, _BM)], xb, sem).wait()
            pltpu.make_async_copy(ykT_hbm.at[:, pl.ds(0, _BM)], yb, sem).wait()

        pltpu.make_async_copy(nT_hbm.at[pl.ds(0, _BM), :], nb, sem).wait()

    XT = XT_ref[...]
    YT = YT_ref[...]

    def _merge(blk_max, blk_arg, blk_any):
        better = blk_max > runmax[...]
        runmax[...] = jnp.where(better, blk_max, runmax[...])
        runidx[...] = jnp.where(better, blk_arg, runidx[...])
        flagacc[...] = jnp.maximum(flagacc[...], blk_any.astype(jnp.float32))

    @pl.when(i == 0)
    def _init():
        runmax[...] = jnp.full((1, _B), _NEG_HUGE, jnp.float32)
        runidx[...] = jnp.zeros((1, _B), jnp.int32)
        flagacc[...] = jnp.zeros((1, 1), jnp.float32)
        _issue(0, xb0, yb0, nb0, sem0)

    @pl.when(i == _NB - 1)
    def _ov_keys():
        xb0[...] = xkov_ref[...]
        yb0[...] = ykov_ref[...]

    colv = lax.broadcasted_iota(jnp.int32, (_BM, 1), 0).astype(jnp.float32)
    off = jnp.minimum(i * _BM, _LAST_OFF)

    def _step(xb, yb, nb, sem, xbn, ybn, nbn, semn):
        @pl.when(i + 1 < _NB)
        def _pf():
            _issue(i + 1, xbn, ybn, nbn, semn)

        _wait(i, xb, yb, nb, sem)
        blk_max, blk_arg, blk_any = _score_body(XT, YT, xb[...], yb[...],
                                                nb[...], colv)
        _merge(blk_max, blk_arg + off, blk_any)

    @pl.when(i % 2 == 0)
    def _even():
        _step(xb0, yb0, nb0, sem0, xb1, yb1, nb1, sem1)

    @pl.when(i % 2 == 1)
    def _odd():
        _step(xb1, yb1, nb1, sem1, xb0, yb0, nb0, sem0)

    @pl.when(i == _NB - 1)
    def _fin():
        idx_out[...] = runidx[...]
        flag_out[...] = flagacc[...]


def _scores_argmax(X, Y, x_keys, y_keys, noise):
    XT = X.T
    YT = Y.T
    xkT = x_keys.T
    ykT = y_keys.T
    nT = noise.T
    xk_ov = lax.slice(x_keys, (_LAST_OFF, 0), (_M, _DK)).T
    yk_ov = lax.slice(y_keys, (_LAST_OFF, 0), (_M, _DK)).T
    idx, flag = pl.pallas_call(
        _score_kernel,
        grid=(_NB,),
        in_specs=[
            pl.BlockSpec((_DK, _B), lambda i: (0, 0)),
            pl.BlockSpec((_DK, _B), lambda i: (0, 0)),
            pl.BlockSpec((_DK, _BM), lambda i: (0, 0)),
            pl.BlockSpec((_DK, _BM), lambda i: (0, 0)),
            pl.BlockSpec(memory_space=pl.ANY),
            pl.BlockSpec(memory_space=pl.ANY),
            pl.BlockSpec(memory_space=pl.ANY),
        ],
        out_specs=[
            pl.BlockSpec((1, _B), lambda i: (0, 0)),
            pl.BlockSpec((1, 1), lambda i: (0, 0)),
        ],
        out_shape=[
            jax.ShapeDtypeStruct((1, _B), jnp.int32),
            jax.ShapeDtypeStruct((1, 1), jnp.float32),
        ],
        scratch_shapes=[
            pltpu.VMEM((1, _B), jnp.float32),
            pltpu.VMEM((1, _B), jnp.int32),
            pltpu.VMEM((1, 1), jnp.float32),
            pltpu.VMEM((_DK, _BM), jnp.float32),
            pltpu.VMEM((_DK, _BM), jnp.float32),
            pltpu.VMEM((_DK, _BM), jnp.float32),
            pltpu.VMEM((_DK, _BM), jnp.float32),
            pltpu.VMEM((_BM, _B), jnp.float32),
            pltpu.VMEM((_BM, _B), jnp.float32),
            pltpu.SemaphoreType.DMA,
            pltpu.SemaphoreType.DMA,
        ],
    )(XT, YT, xk_ov, yk_ov, xkT, ykT, nT)
    return idx, flag


def _make_sc_gather():
    info = plsc.get_sparse_core_info()
    nw = info.num_cores * info.num_subcores
    b_per_w = _B // nw
    mesh = plsc.VectorSubcoreMesh(core_axis_name="c", subcore_axis_name="s")

    @functools.partial(
        pl.kernel, mesh=mesh,
        out_type=jax.ShapeDtypeStruct((_B, 2 * _DOUT), jnp.float32),
        scratch_types=[
            pltpu.VMEM((b_per_w,), jnp.int32),
            pltpu.VMEM((b_per_w, 2 * _DOUT), jnp.float32),
            pltpu.SemaphoreType.DMA,
        ],
    )
    def _gather(table_hbm, idx_hbm, out_hbm, idx_v, rows_v, sem):
        wid = lax.axis_index("s") * info.num_cores + lax.axis_index("c")
        base = wid * b_per_w
        pltpu.sync_copy(idx_hbm.at[pl.ds(base, b_per_w)], idx_v)
        pltpu.async_copy(table_hbm.at[idx_v], rows_v, sem).wait()
        pltpu.sync_copy(rows_v, out_hbm.at[pl.ds(base, b_per_w)])

    return _gather


def kernel(X, Y, x_keys, y_keys, values, noise):
    idx, flag = _scores_argmax(X, Y, x_keys, y_keys, noise)
    idx = idx.reshape(_B)
    gather = _make_sc_gather()
    pair = gather(values.reshape(_M // 2, 2 * _DOUT), jnp.right_shift(idx, 1))
    x_hat = jnp.where((idx & 1)[:, None] == 1, pair[:, _DOUT:], pair[:, :_DOUT])
    return jnp.where(flag[0, 0] > 0.0, x_hat, jnp.zeros_like(x_hat))


# speedup vs baseline: 1.1027x; 1.0077x over previous
import functools

import jax
import jax.numpy as jnp
from jax import lax
from jax.experimental import pallas as pl
from jax.experimental.pallas import tpu as pltpu
from jax.experimental.pallas import tpu_sc as plsc

_M = 100000
_B = 1024
_DK = 64
_DOUT = 64
_THRESH = 0.1
_BM = 1024
_NB = _M // _BM + 1
_LAST_OFF = _M - _BM

_NEG_HUGE = -3.0e38


def _score_body(XT, YT, xkb, ykb, nt, colv):
    dn = (((0,), (0,)), ((), ()))
    kq_x = lax.dot_general(xkb, XT, dn, preferred_element_type=jnp.float32)
    kq_y = lax.dot_general(ykb, YT, dn, preferred_element_type=jnp.float32)
    ones = jnp.ones((_DK, 1), jnp.float32)
    yn_col = lax.dot_general(ykb * ykb, ones, dn,
                             preferred_element_type=jnp.float32,
                             precision=lax.Precision.HIGHEST)
    ones1 = jnp.ones((1, _DK), jnp.float32)
    y2_row = lax.dot_general(ones1, YT * YT, (((1,), (0,)), ((), ())),
                             preferred_element_type=jnp.float32,
                             precision=lax.Precision.HIGHEST)
    ynh_col = 0.5 * yn_col
    y2h_row = 0.5 * y2_row - (0.5 * _THRESH)
    gate = (kq_y - ynh_col > y2h_row) & (kq_x > 0.0)
    scores = jnp.where(gate, (kq_x + kq_y) + nt, nt)
    blk_max = jnp.max(scores, axis=0, keepdims=True)
    cand = jnp.where(scores == blk_max, colv, 3.0e38)
    blk_arg = jnp.min(cand, axis=0, keepdims=True).astype(jnp.int32)
    blk_any = jnp.any(gate)
    return blk_max, blk_arg, blk_any


def _score_kernel(XT_ref, YT_ref, xkov_ref, ykov_ref, xkT_hbm, ykT_hbm, nT_hbm,
                  idx_out, flag_out,
                  runmax, runidx, flagacc,
                  xb0, xb1, yb0, yb1, nb0, nb1, sem0, sem1):
    i = pl.program_id(0)

    def _issue(k, xb, yb, nb, sem):
        k = jnp.asarray(k, jnp.int32)
        off = pl.multiple_of(jnp.minimum(k * _BM, _LAST_OFF), 8)

        @pl.when(k < _NB - 1)
        def _keys():
            pltpu.make_async_copy(xkT_hbm.at[:, pl.ds(k * _BM, _BM)], xb,
                                  sem).start()
            pltpu.make_async_copy(ykT_hbm.at[:, pl.ds(k * _BM, _BM)], yb,
                                  sem).start()

        pltpu.make_async_copy(nT_hbm.at[pl.ds(off, _BM), :], nb, sem).start()

    def _wait(k, xb, yb, nb, sem):
        @pl.when(k < _NB - 1)
        def _keys():
            pltpu.make_async_copy(xkT_hbm.at[:, pl.ds(0, _BM)], xb, sem).wait()
            pltpu.make_async_copy(ykT_hbm.at[:, pl.ds(0, _BM)], yb, sem).wait()

        pltpu.make_async_copy(nT_hbm.at[pl.ds(0, _BM), :], nb, sem).wait()

    XT = XT_ref[...]
    YT = YT_ref[...]

    def _merge(blk_max, blk_arg, blk_any):
        better = blk_max > runmax[...]
        runmax[...] = jnp.where(better, blk_max, runmax[...])
        runidx[...] = jnp.where(better, blk_arg, runidx[...])
        flagacc[...] = jnp.maximum(flagacc[...], blk_any.astype(jnp.float32))

    @pl.when(i == 0)
    def _init():
        runmax[...] = jnp.full((1, _B), _NEG_HUGE, jnp.float32)
        runidx[...] = jnp.zeros((1, _B), jnp.int32)
        flagacc[...] = jnp.zeros((1, 1), jnp.float32)
        _issue(0, xb0, yb0, nb0, sem0)

    @pl.when(i == _NB - 1)
    def _ov_keys():
        xb0[...] = xkov_ref[...]
        yb0[...] = ykov_ref[...]

    colv = lax.broadcasted_iota(jnp.int32, (_BM, 1), 0).astype(jnp.float32)
    off = jnp.minimum(i * _BM, _LAST_OFF)

    def _step(xb, yb, nb, sem, xbn, ybn, nbn, semn):
        @pl.when(i + 1 < _NB)
        def _pf():
            _issue(i + 1, xbn, ybn, nbn, semn)

        _wait(i, xb, yb, nb, sem)
        blk_max, blk_arg, blk_any = _score_body(XT, YT, xb[...], yb[...],
                                                nb[...], colv)
        _merge(blk_max, blk_arg + off, blk_any)

    @pl.when(i % 2 == 0)
    def _even():
        _step(xb0, yb0, nb0, sem0, xb1, yb1, nb1, sem1)

    @pl.when(i % 2 == 1)
    def _odd():
        _step(xb1, yb1, nb1, sem1, xb0, yb0, nb0, sem0)

    @pl.when(i == _NB - 1)
    def _fin():
        idx_out[...] = runidx[...]
        flag_out[...] = flagacc[...]


def _scores_argmax(X, Y, x_keys, y_keys, noise):
    XT = X.T
    YT = Y.T
    xkT = x_keys.T
    ykT = y_keys.T
    nT = noise.T
    xk_ov = lax.slice(x_keys, (_LAST_OFF, 0), (_M, _DK)).T
    yk_ov = lax.slice(y_keys, (_LAST_OFF, 0), (_M, _DK)).T
    idx, flag = pl.pallas_call(
        _score_kernel,
        grid=(_NB,),
        in_specs=[
            pl.BlockSpec((_DK, _B), lambda i: (0, 0)),
            pl.BlockSpec((_DK, _B), lambda i: (0, 0)),
            pl.BlockSpec((_DK, _BM), lambda i: (0, 0)),
            pl.BlockSpec((_DK, _BM), lambda i: (0, 0)),
            pl.BlockSpec(memory_space=pl.ANY),
            pl.BlockSpec(memory_space=pl.ANY),
            pl.BlockSpec(memory_space=pl.ANY),
        ],
        out_specs=[
            pl.BlockSpec((1, _B), lambda i: (0, 0)),
            pl.BlockSpec((1, 1), lambda i: (0, 0)),
        ],
        out_shape=[
            jax.ShapeDtypeStruct((1, _B), jnp.int32),
            jax.ShapeDtypeStruct((1, 1), jnp.float32),
        ],
        scratch_shapes=[
            pltpu.VMEM((1, _B), jnp.float32),
            pltpu.VMEM((1, _B), jnp.int32),
            pltpu.VMEM((1, 1), jnp.float32),
            pltpu.VMEM((_DK, _BM), jnp.float32),
            pltpu.VMEM((_DK, _BM), jnp.float32),
            pltpu.VMEM((_DK, _BM), jnp.float32),
            pltpu.VMEM((_DK, _BM), jnp.float32),
            pltpu.VMEM((_BM, _B), jnp.float32),
            pltpu.VMEM((_BM, _B), jnp.float32),
            pltpu.SemaphoreType.DMA,
            pltpu.SemaphoreType.DMA,
        ],
    )(XT, YT, xk_ov, yk_ov, xkT, ykT, nT)
    return idx, flag


def _make_sc_gather():
    info = plsc.get_sparse_core_info()
    nw = info.num_cores * info.num_subcores
    b_per_w = _B // nw
    mesh = plsc.VectorSubcoreMesh(core_axis_name="c", subcore_axis_name="s")

    @functools.partial(
        pl.kernel, mesh=mesh,
        out_type=jax.ShapeDtypeStruct((_B, 2 * _DOUT), jnp.float32),
        scratch_types=[
            pltpu.VMEM((b_per_w,), jnp.int32),
            pltpu.VMEM((b_per_w, 2 * _DOUT), jnp.float32),
            pltpu.SemaphoreType.DMA,
        ],
    )
    def _gather(table_hbm, idx_hbm, out_hbm, idx_v, rows_v, sem):
        wid = lax.axis_index("s") * info.num_cores + lax.axis_index("c")
        base = wid * b_per_w
        pltpu.sync_copy(idx_hbm.at[pl.ds(base, b_per_w)], idx_v)
        pltpu.async_copy(table_hbm.at[idx_v], rows_v, sem).wait()
        pltpu.sync_copy(rows_v, out_hbm.at[pl.ds(base, b_per_w)])

    return _gather


def kernel(X, Y, x_keys, y_keys, values, noise):
    idx, flag = _scores_argmax(X, Y, x_keys, y_keys, noise)
    idx = idx.reshape(_B)
    gather = _make_sc_gather()
    pair = gather(values.reshape(_M // 2, 2 * _DOUT), jnp.right_shift(idx, 1))
    x_hat = jnp.where((idx & 1)[:, None] == 1, pair[:, _DOUT:], pair[:, :_DOUT])
    return jnp.where(flag[0, 0] > 0.0, x_hat, jnp.zeros_like(x_hat))
